# baseline (speedup 1.0000x reference)
import jax
import jax.numpy as jnp
from jax import lax
from jax.experimental import pallas as pl
from jax.experimental.pallas import tpu as pltpu
from jax.experimental.pallas import tpu_sc as plsc

SEQ = 200
BATCH = 4096
DIM = 32
VOCAB = 1000000
NC, NS = 2, 16
NW = NC * NS
BPW = BATCH // NW
NB = 20

TBLK = 32768
SUB8 = TBLK // 8
LB = TBLK.bit_length() - 1
S8B = SUB8.bit_length() - 1
TGRID = (VOCAB + TBLK - 1) // TBLK
VROWS8 = TGRID * SUB8
VPAD8 = VROWS8 * 8


def _sc_body(sent_hbm, table_hbm, out_hbm, idx_v, outb, *scr):
    bufs = scr[:NB]
    sems = scr[NB:]
    wid = lax.axis_index("s") * NC + lax.axis_index("c")
    base = wid * BPW

    pltpu.sync_copy(sent_hbm.at[:, pl.ds(base, BPW)], idx_v)

    def pbody(t, c):
        for j in range(BPW // 16):
            v = idx_v[t, pl.ds(j * 16, 16)]
            idx_v[t, pl.ds(j * 16, 16)] = (
                ((v >> LB) << LB) | ((v & (SUB8 - 1)) << 3) | ((v >> S8B) & 7))
        return c
    lax.fori_loop(0, SEQ, pbody, 0)

    for b in range(NB):
        pltpu.async_copy(table_hbm.at[idx_v.at[b]], bufs[b], sems[b])

    def step(k, c):
        t = NB + NB * k
        for b in range(NB):
            pltpu.make_async_copy(table_hbm.at[idx_v.at[0]], bufs[b], sems[b]).wait()
            pltpu.async_copy(table_hbm.at[idx_v.at[t + b]], bufs[b], sems[b],
                             add=True)
        return c
    lax.fori_loop(0, (SEQ - NB) // NB, step, 0)

    for b in range(NB):
        pltpu.make_async_copy(table_hbm.at[idx_v.at[0]], bufs[b], sems[b]).wait()

    def tree_sum(vals):
        while len(vals) > 1:
            vals = [vals[j] + vals[j + 1] for j in range(0, len(vals) - 1, 2)] \
                + ([vals[-1]] if len(vals) % 2 else [])
        return vals[0]

    def fbody(i, c):
        s = tree_sum([buf[i, pl.ds(0, 2 * 16)] for buf in bufs])
        outb[i, pl.ds(0, 2 * 16)] = s
        return c
    lax.fori_loop(0, BPW, fbody, 0, unroll=4)

    pltpu.sync_copy(outb, out_hbm.at[pl.ds(base, BPW), :])


def _tc_pack_body(x_ref, o_ref):
    lo = x_ref[0:16, :].astype(jnp.bfloat16).astype(jnp.float32)
    hi = x_ref[16:32, :].astype(jnp.bfloat16).astype(jnp.float32)
    ulo = lax.bitcast_convert_type(lo, jnp.uint32) >> 16
    uhi = lax.bitcast_convert_type(hi, jnp.uint32) & jnp.uint32(0xFFFF0000)
    w = lax.bitcast_convert_type(ulo | uhi, jnp.float32)
    y = jnp.concatenate(
        [w[:, q * SUB8:(q + 1) * SUB8] for q in range(8)], axis=0)
    o_ref[...] = y.T


def _relayout_table(table):
    tableT = table.T
    tbl = pl.pallas_call(
        _tc_pack_body,
        grid=(TGRID,),
        in_specs=[pl.BlockSpec((DIM, TBLK), lambda i: (0, i))],
        out_specs=pl.BlockSpec((SUB8, 128), lambda i: (i, 0)),
        out_shape=jax.ShapeDtypeStruct((VROWS8, 128), jnp.float32),
    )(tableT)
    tbl_bf = lax.bitcast_convert_type(tbl, jnp.bfloat16)
    return tbl_bf.reshape(VPAD8, DIM)


def _tc_unpack_body(z_ref, o_ref):
    inv = jnp.float32(1.0 / SEQ)
    w = lax.bitcast_convert_type(z_ref[...], jnp.int32)
    lo = lax.bitcast_convert_type(w << 16, jnp.float32)
    hi = lax.bitcast_convert_type(w & jnp.int32(-65536), jnp.float32)
    o_ref[...] = jnp.concatenate([lo, hi], axis=1) * inv


def _unpack_scale(out_sc):
    z = lax.bitcast_convert_type(
        out_sc.reshape(BATCH, DIM // 2, 2), jnp.float32)
    return pl.pallas_call(
        _tc_unpack_body,
        out_shape=jax.ShapeDtypeStruct((BATCH, DIM), jnp.float32),
    )(z)


def kernel(sentence, table):
    k = pl.kernel(
        _sc_body,
        out_type=jax.ShapeDtypeStruct((BATCH, DIM), jnp.bfloat16),
        mesh=plsc.VectorSubcoreMesh(core_axis_name="c", subcore_axis_name="s"),
        compiler_params=pltpu.CompilerParams(use_tc_tiling_on_sc=False),
        scratch_types=(
            [pltpu.VMEM((SEQ, BPW), jnp.int32),
             pltpu.VMEM((BPW, DIM), jnp.bfloat16)]
            + [pltpu.VMEM((BPW, DIM), jnp.bfloat16)] * NB
            + [pltpu.SemaphoreType.DMA] * NB
        ),
    )
    return _unpack_scale(k(sentence, _relayout_table(table)))


# speedup vs baseline: 96.0188x; 96.0188x over previous
import jax
import jax.numpy as jnp
from jax import lax
from jax.experimental import pallas as pl
from jax.experimental.pallas import tpu as pltpu
from jax.experimental.pallas import tpu_sc as plsc

SEQ = 200
BATCH = 4096
DIM = 32
VOCAB = 1000000
NC, NS = 2, 16
NW = NC * NS
BPW = BATCH // NW
NB = 8
TROW = SEQ // NB

TBLK = 32768
SUB = TBLK // 4
LB = TBLK.bit_length() - 1
SB = SUB.bit_length() - 1
TGRID = (VOCAB + TBLK - 1) // TBLK
VROWS = TGRID * SUB
VPAD = VROWS * 4


def _sc_body(sent_hbm, table_hbm, out_hbm, idx_v, *scr):
    bufs = scr[:NB]
    sems = scr[NB:]
    wid = lax.axis_index("s") * NC + lax.axis_index("c")

    pltpu.sync_copy(sent_hbm.at[:, wid], idx_v)

    def pbody(t, c):
        for r in range(NB):
            for j in range(BPW // 16):
                v = idx_v[t, r, pl.ds(j * 16, 16)]
                idx_v[t, r, pl.ds(j * 16, 16)] = (
                    ((v >> LB) << LB) | ((v & (SUB - 1)) << 2)
                    | ((v >> SB) & 3))
        return c
    lax.fori_loop(0, TROW, pbody, 0)

    for b in range(NB):
        pltpu.async_copy(table_hbm.at[idx_v.at[0, b]], bufs[b], sems[b])

    def step(k, c):
        for b in range(NB):
            pltpu.make_async_copy(table_hbm.at[idx_v.at[0, 0]], bufs[b],
                                  sems[b]).wait()
            pltpu.async_copy(table_hbm.at[idx_v.at[k + 1, b]], bufs[b],
                             sems[b], add=True)
        return c
    lax.fori_loop(0, TROW - 1, step, 0)

    for b in range(NB):
        pltpu.make_async_copy(table_hbm.at[idx_v.at[0, 0]], bufs[b],
                              sems[b]).wait()

    inv = jnp.float32(1.0 / SEQ)

    def tree_sum(vals):
        while len(vals) > 1:
            vals = [vals[j] + vals[j + 1] for j in range(0, len(vals) - 1, 2)] \
                + ([vals[-1]] if len(vals) % 2 else [])
        return vals[0]

    def fbody(i, c):
        for off in (0, 16):
            s = tree_sum([buf[i, pl.ds(off, 16)] for buf in bufs])
            bufs[0][i, pl.ds(off, 16)] = s * inv
        return c
    lax.fori_loop(0, BPW, fbody, 0, unroll=8)

    pltpu.sync_copy(bufs[0], out_hbm.at[pl.ds(wid * BPW, BPW), :])


def _tc_transpose_body(x_ref, o_ref):
    y = jnp.concatenate(
        [x_ref[:, q * SUB:(q + 1) * SUB] for q in range(4)], axis=0)
    o_ref[...] = y.T


def _relayout_table(table):
    tableT = table.T
    tbl = pl.pallas_call(
        _tc_transpose_body,
        grid=(TGRID,),
        in_specs=[pl.BlockSpec((DIM, TBLK), lambda i: (0, i))],
        out_specs=pl.BlockSpec((SUB, 128), lambda i: (i, 0)),
        out_shape=jax.ShapeDtypeStruct((VROWS, 128), jnp.float32),
    )(tableT)
    return tbl.reshape(VPAD, DIM)


def kernel(sentence, table):
    sent4 = sentence.reshape(TROW, NB, NW, BPW).transpose(0, 2, 1, 3)
    k = pl.kernel(
        _sc_body,
        out_type=jax.ShapeDtypeStruct((BATCH, DIM), jnp.float32),
        mesh=plsc.VectorSubcoreMesh(core_axis_name="c", subcore_axis_name="s"),
        compiler_params=pltpu.CompilerParams(use_tc_tiling_on_sc=False),
        scratch_types=(
            [pltpu.VMEM((TROW, NB, BPW), jnp.int32)]
            + [pltpu.VMEM((BPW, DIM), jnp.float32)] * NB
            + [pltpu.SemaphoreType.DMA] * NB
        ),
    )
    return k(sent4, _relayout_table(table))
